# baseline (speedup 1.0000x reference)
import functools

import jax
import jax.numpy as jnp
import numpy as np
from jax import lax
from jax.experimental import pallas as pl
from jax.experimental.pallas import tpu as pltpu
from jax.experimental.pallas import tpu_sc as plsc

F = 26
FD = 4000
D = 16
B = 4096
TBL = F * FD
FF = F * F

NC = 2
NS = 16
NW = NC * NS
BPW = B // NW
CB = 4
NCHUNK = BPW // CB
SUB = CB * F
ROWS = F * SUB

_PI, _PJ = np.triu_indices(F, k=1)
_OFF_A = [int(j) * SUB + int(i) for i, j in zip(_PI, _PJ)]
_OFF_B = [int(i) * SUB + int(j) for i, j in zip(_PI, _PJ)]


RB = 1024
NBK = TBL // RB
TAIL = TBL - NBK * RB
UNITS = F * NBK
UPW = UNITS // NW
EXTRA = UNITS - UPW * NW


UPW2 = 2 * (-(-UNITS // (2 * NW)))


def _tr_body(wt_hbm, wtail_hbm, w1d_hbm, tin0, tin1, tout0, tout1,
             sem_i0, sem_i1, sem_o0, sem_o1):
    cid = lax.axis_index("c")
    sid = lax.axis_index("s")
    wid = sid * NC + cid
    lanes = lax.iota(jnp.int32, D)
    lanes_d = lanes * D
    lo = wid * UPW2
    last = UNITS - 1

    tins = (tin0, tin1)
    touts = (tout0, tout1)
    sis = (sem_i0, sem_i1)
    sos = (sem_o0, sem_o1)

    def src(i):
        gb = jnp.minimum(lo + i, last)
        t = gb // NBK
        r0 = (gb - t * NBK) * RB
        return t, r0, wt_hbm.at[t, :, pl.ds(r0, RB)]

    for q in range(2):
        pltpu.async_copy(src(q)[2], tins[q], sis[q])

    def pair(p, carry):
        for q in range(2):
            i = 2 * p + q
            tin, tout, si, so = tins[q], touts[q], sis[q], sos[q]
            t, r0, _ = src(i)
            pltpu.make_async_copy(wt_hbm.at[0, :, pl.ds(0, RB)], tin, si).wait()

            @pl.when(p > 0)
            def _():
                pltpu.make_async_copy(
                    tout, w1d_hbm.at[pl.ds(0, RB * D)], so
                ).wait()

            def dscan(d, c2):
                base = lanes_d + d
                for rc in range(RB // D):
                    v = tin[d, pl.ds(rc * D, D)]
                    plsc.store_scatter(tout, [base + rc * D * D], v)
                return c2

            lax.fori_loop(0, D, dscan, 0, unroll=False)
            pltpu.async_copy(
                tout, w1d_hbm.at[pl.ds((t * TBL + r0) * D, RB * D)], so
            )
            _, _, nsrc = src(i + 2)
            pltpu.async_copy(nsrc, tin, si)
        return carry

    lax.fori_loop(0, UPW2 // 2, pair, 0, unroll=False)

    for q in range(2):
        pltpu.make_async_copy(
            wt_hbm.at[0, :, pl.ds(0, RB)], tins[q], sis[q]
        ).wait()
        pltpu.make_async_copy(
            touts[q], w1d_hbm.at[pl.ds(0, RB * D)], sos[q]
        ).wait()

    @pl.when(wid < F)
    def _():
        t = wid
        n = TAIL * D
        pltpu.sync_copy(wtail_hbm.at[pl.ds(t * n, n)], tout0.at[pl.ds(0, n)])
        pltpu.sync_copy(
            tout0.at[pl.ds(0, n)],
            w1d_hbm.at[pl.ds((t * TBL + NBK * RB) * D, n)],
        )


@functools.cache
def _tr_kernel():
    return pl.kernel(
        _tr_body,
        out_type=jax.ShapeDtypeStruct((F * TBL * D,), jnp.float32),
        mesh=plsc.VectorSubcoreMesh(
            core_axis_name="c", subcore_axis_name="s",
            num_cores=NC, num_subcores=NS,
        ),
        compiler_params=pltpu.CompilerParams(
            needs_layout_passes=False, use_tc_tiling_on_sc=True
        ),
        scratch_types=[
            pltpu.VMEM((D, RB), jnp.float32),
            pltpu.VMEM((D, RB), jnp.float32),
            pltpu.VMEM((RB * D,), jnp.float32),
            pltpu.VMEM((RB * D,), jnp.float32),
            pltpu.SemaphoreType.DMA,
            pltpu.SemaphoreType.DMA,
            pltpu.SemaphoreType.DMA,
            pltpu.SemaphoreType.DMA,
        ],
    )


def _sc_body(w_hbm, x_hbm, out_hbm, x_v, idx_v, rows_v, res_v, out_v, sem):
    cid = lax.axis_index("c")
    sid = lax.axis_index("s")
    wid = sid * NC + cid
    b0 = wid * BPW

    pltpu.sync_copy(x_hbm.at[pl.ds(b0 * F, BPW * F)], x_v.at[pl.ds(0, BPW * F)])

    lanes = lax.iota(jnp.int32, D)
    f_lo = lanes * FD
    f_hi = (lanes + D) * FD

    def chunk_body(c, carry):
        def bl_body(bl, carry2):
            xoff = (c * CB + bl) * F
            off = bl * F
            idx_v[pl.ds(off, D)] = x_v[pl.ds(xoff, D)] + f_lo
            idx_v[pl.ds(off + D, D)] = x_v[pl.ds(xoff + D, D)] + f_hi
            return carry2

        lax.fori_loop(0, CB, bl_body, 0, unroll=False)

        handles = []
        for t in range(F):
            handles.append(
                pltpu.async_copy(
                    w_hbm.at[t].at[idx_v.at[pl.ds(0, SUB)]],
                    rows_v.at[pl.ds(t * SUB, SUB)],
                    sem,
                )
            )
        for h in handles:
            h.wait()

        def b_body(bl, carry2):
            rb = bl * F
            acc = jnp.zeros((D,), jnp.float32)
            for oa, ob in zip(_OFF_A, _OFF_B):
                acc = acc + rows_v[rb + oa] * rows_v[rb + ob]
            res_v[c * CB + bl] = acc
            return carry2

        lax.fori_loop(0, CB, b_body, 0, unroll=False)
        return carry

    lax.fori_loop(0, NCHUNK, chunk_body, 0, unroll=False)

    for grp in range(BPW // D):
        base = grp * D
        tot = jnp.zeros((D,), jnp.float32)
        for l in range(D):
            s = jnp.sum(res_v[base + l])
            tot = jnp.where(lanes == l, s, tot)
        out_v[pl.ds(base, D)] = tot

    pltpu.sync_copy(out_v, out_hbm.at[pl.ds(b0, BPW)])


@functools.cache
def _ffm_kernel():
    return pl.kernel(
        _sc_body,
        out_type=jax.ShapeDtypeStruct((B,), jnp.float32),
        mesh=plsc.VectorSubcoreMesh(
            core_axis_name="c", subcore_axis_name="s",
            num_cores=NC, num_subcores=NS,
        ),
        compiler_params=pltpu.CompilerParams(
            needs_layout_passes=False, use_tc_tiling_on_sc=False
        ),
        scratch_types=[
            pltpu.VMEM((BPW * F + D,), jnp.int32),
            pltpu.VMEM((SUB + D,), jnp.int32),
            pltpu.VMEM((ROWS, D), jnp.float32),
            pltpu.VMEM((BPW, D), jnp.float32),
            pltpu.VMEM((BPW,), jnp.float32),
            pltpu.SemaphoreType.DMA,
        ],
    )


@jax.jit
def kernel(x, W):
    xflat = x.astype(jnp.int32).reshape(B * F)
    wt = jnp.transpose(W, (0, 2, 1))
    wtail = W[:, NBK * RB:, :].reshape(F * TAIL * D)
    w1d = _tr_kernel()(wt, wtail)
    wsc = w1d.reshape(F, TBL, D)
    return _ffm_kernel()(wsc, xflat)


# speedup vs baseline: 1.0989x; 1.0989x over previous
import functools

import jax
import jax.numpy as jnp
import numpy as np
from jax import lax
from jax.experimental import pallas as pl
from jax.experimental.pallas import tpu as pltpu
from jax.experimental.pallas import tpu_sc as plsc

F = 26
FD = 4000
D = 16
B = 4096
TBL = F * FD
FF = F * F

NC = 2
NS = 16
NW = NC * NS
BPW = B // NW
CB = 4
NCHUNK = BPW // CB
SUB = CB * F
ROWS = F * SUB

_PI, _PJ = np.triu_indices(F, k=1)
_OFF_A = [int(j) * SUB + int(i) for i, j in zip(_PI, _PJ)]
_OFF_B = [int(i) * SUB + int(j) for i, j in zip(_PI, _PJ)]


RB = 512
NBK = TBL // RB
TAIL = TBL - NBK * RB
UNITS = F * NBK
UPW = UNITS // NW
EXTRA = UNITS - UPW * NW


UPW2 = 2 * (-(-UNITS // (2 * NW)))


def _tr_body(wt_hbm, wtail_hbm, w1d_hbm, tin0, tin1, tout0, tout1,
             sem_i0, sem_i1, sem_o0, sem_o1):
    cid = lax.axis_index("c")
    sid = lax.axis_index("s")
    wid = sid * NC + cid
    lanes = lax.iota(jnp.int32, D)
    lanes_d = lanes * D
    lo = wid * UPW2
    last = UNITS - 1

    tins = (tin0, tin1)
    touts = (tout0, tout1)
    sis = (sem_i0, sem_i1)
    sos = (sem_o0, sem_o1)

    def src(i):
        gb = jnp.minimum(lo + i, last)
        t = gb // NBK
        r0 = (gb - t * NBK) * RB
        return t, r0, wt_hbm.at[t, :, pl.ds(r0, RB)]

    for q in range(2):
        pltpu.async_copy(src(q)[2], tins[q], sis[q])

    def pair(p, carry):
        for q in range(2):
            i = 2 * p + q
            tin, tout, si, so = tins[q], touts[q], sis[q], sos[q]
            t, r0, _ = src(i)
            pltpu.make_async_copy(wt_hbm.at[0, :, pl.ds(0, RB)], tin, si).wait()

            @pl.when(p > 0)
            def _():
                pltpu.make_async_copy(
                    tout, w1d_hbm.at[pl.ds(0, RB * D)], so
                ).wait()

            def dscan(d, c2):
                base = lanes_d + d
                for rc in range(RB // D):
                    v = tin[d, pl.ds(rc * D, D)]
                    plsc.store_scatter(tout, [base + rc * D * D], v)
                return c2

            lax.fori_loop(0, D, dscan, 0, unroll=False)
            pltpu.async_copy(
                tout, w1d_hbm.at[pl.ds((t * TBL + r0) * D, RB * D)], so
            )
            _, _, nsrc = src(i + 2)
            pltpu.async_copy(nsrc, tin, si)
        return carry

    lax.fori_loop(0, UPW2 // 2, pair, 0, unroll=False)

    for q in range(2):
        pltpu.make_async_copy(
            wt_hbm.at[0, :, pl.ds(0, RB)], tins[q], sis[q]
        ).wait()
        pltpu.make_async_copy(
            touts[q], w1d_hbm.at[pl.ds(0, RB * D)], sos[q]
        ).wait()

    @pl.when(wid < F)
    def _():
        t = wid
        n = TAIL * D
        pltpu.sync_copy(wtail_hbm.at[pl.ds(t * n, n)], tout0.at[pl.ds(0, n)])
        pltpu.sync_copy(
            tout0.at[pl.ds(0, n)],
            w1d_hbm.at[pl.ds((t * TBL + NBK * RB) * D, n)],
        )


@functools.cache
def _tr_kernel():
    return pl.kernel(
        _tr_body,
        out_type=jax.ShapeDtypeStruct((F * TBL * D,), jnp.float32),
        mesh=plsc.VectorSubcoreMesh(
            core_axis_name="c", subcore_axis_name="s",
            num_cores=NC, num_subcores=NS,
        ),
        compiler_params=pltpu.CompilerParams(
            needs_layout_passes=False, use_tc_tiling_on_sc=True
        ),
        scratch_types=[
            pltpu.VMEM((D, RB), jnp.float32),
            pltpu.VMEM((D, RB), jnp.float32),
            pltpu.VMEM((RB * D,), jnp.float32),
            pltpu.VMEM((RB * D,), jnp.float32),
            pltpu.SemaphoreType.DMA,
            pltpu.SemaphoreType.DMA,
            pltpu.SemaphoreType.DMA,
            pltpu.SemaphoreType.DMA,
        ],
    )


def _sc_body(w_hbm, x_hbm, out_hbm, x_v, idx_v0, idx_v1, rows_v0, rows_v1,
             res_v, out_v, sem0, sem1):
    cid = lax.axis_index("c")
    sid = lax.axis_index("s")
    wid = sid * NC + cid
    b0 = wid * BPW

    pltpu.sync_copy(x_hbm.at[pl.ds(b0 * F, BPW * F)], x_v.at[pl.ds(0, BPW * F)])

    lanes = lax.iota(jnp.int32, D)
    f_lo = lanes * FD
    f_hi = (lanes + D) * FD

    idxs = (idx_v0, idx_v1)
    rows = (rows_v0, rows_v1)
    sems = (sem0, sem1)

    def build_and_fire(c, q):
        idx_v, rows_v, sem = idxs[q], rows[q], sems[q]

        def bl_body(bl, carry2):
            xoff = (c * CB + bl) * F
            off = bl * F
            idx_v[pl.ds(off, D)] = x_v[pl.ds(xoff, D)] + f_lo
            idx_v[pl.ds(off + D, D)] = x_v[pl.ds(xoff + D, D)] + f_hi
            return carry2

        lax.fori_loop(0, CB, bl_body, 0, unroll=False)
        for t in range(F):
            pltpu.async_copy(
                w_hbm.at[t].at[idx_v.at[pl.ds(0, SUB)]],
                rows_v.at[pl.ds(t * SUB, SUB)],
                sem,
            )

    def drain(q):
        for t in range(F):
            pltpu.make_async_copy(
                w_hbm.at[0].at[idxs[q].at[pl.ds(0, SUB)]],
                rows[q].at[pl.ds(t * SUB, SUB)],
                sems[q],
            ).wait()

    build_and_fire(0, 0)

    def pair(p, carry):
        for q in range(2):
            c = 2 * p + q
            rows_v = rows[q]
            drain(q)
            build_and_fire(jnp.minimum(c + 1, NCHUNK - 1), 1 - q)

            def b_body(bl, carry2):
                rb = bl * F
                acc = jnp.zeros((D,), jnp.float32)
                for oa, ob in zip(_OFF_A, _OFF_B):
                    acc = acc + rows_v[rb + oa] * rows_v[rb + ob]
                res_v[c * CB + bl] = acc
                return carry2

            lax.fori_loop(0, CB, b_body, 0, unroll=False)
        return carry

    lax.fori_loop(0, NCHUNK // 2, pair, 0, unroll=False)
    drain(0)

    for grp in range(BPW // D):
        base = grp * D
        tot = jnp.zeros((D,), jnp.float32)
        for l in range(D):
            s = jnp.sum(res_v[base + l])
            tot = jnp.where(lanes == l, s, tot)
        out_v[pl.ds(base, D)] = tot

    pltpu.sync_copy(out_v, out_hbm.at[pl.ds(b0, BPW)])


@functools.cache
def _ffm_kernel():
    return pl.kernel(
        _sc_body,
        out_type=jax.ShapeDtypeStruct((B,), jnp.float32),
        mesh=plsc.VectorSubcoreMesh(
            core_axis_name="c", subcore_axis_name="s",
            num_cores=NC, num_subcores=NS,
        ),
        compiler_params=pltpu.CompilerParams(
            needs_layout_passes=False, use_tc_tiling_on_sc=False
        ),
        scratch_types=[
            pltpu.VMEM((BPW * F + D,), jnp.int32),
            pltpu.VMEM((SUB + D,), jnp.int32),
            pltpu.VMEM((SUB + D,), jnp.int32),
            pltpu.VMEM((ROWS, D), jnp.float32),
            pltpu.VMEM((ROWS, D), jnp.float32),
            pltpu.VMEM((BPW, D), jnp.float32),
            pltpu.VMEM((BPW,), jnp.float32),
            pltpu.SemaphoreType.DMA,
            pltpu.SemaphoreType.DMA,
        ],
    )


@jax.jit
def kernel(x, W):
    xflat = x.astype(jnp.int32).reshape(B * F)
    wt = jnp.transpose(W, (0, 2, 1))
    wtail = W[:, NBK * RB:, :].reshape(F * TAIL * D)
    w1d = _tr_kernel()(wt, wtail)
    wsc = w1d.reshape(F, TBL, D)
    return _ffm_kernel()(wsc, xflat)
